# Initial kernel scaffold; baseline (speedup 1.0000x reference)
import jax
import jax.numpy as jnp
from jax.experimental import pallas as pl


def kernel(user_id, user_age, user_gender, movie_id, movie_categories, emb_users, emb_movies, emb_movie_cats, emb_age, emb_gender, bias_user, bias_movie, user_fc_w, user_fc_b, movie_fc_w, movie_fc_b):
    raise NotImplementedError("write your pallas kernel here")


# speedup vs baseline: 1.5602x; 1.5602x over previous
import functools

import jax
import jax.numpy as jnp
from jax import lax
from jax.experimental import pallas as pl
from jax.experimental.pallas import tpu as pltpu
from jax.experimental.pallas import tpu_sc as plsc

B = 16384
U_DIM = 64
M_DIM = 64

_NC, _NS = 2, 16
NW = _NC * _NS
BPW = B // NW


def _sc_gather_body(uid_h, mid_h, uid16_h, mid16_h, eu_h, em_h, bu16_h, bm16_h,
                    out_u, out_m, out_bu, out_bm,
                    uidx, midx, uidx16, midx16, urows, mrows, buv, bmv, sem):
    wid = lax.axis_index("s") * _NC + lax.axis_index("c")
    base = wid * BPW
    pltpu.sync_copy(uid_h.at[pl.ds(base, BPW)], uidx)
    pltpu.sync_copy(mid_h.at[pl.ds(base, BPW)], midx)
    pltpu.sync_copy(uid16_h.at[pl.ds(base, BPW)], uidx16)
    pltpu.sync_copy(mid16_h.at[pl.ds(base, BPW)], midx16)
    c1 = pltpu.async_copy(eu_h.at[uidx], urows, sem)
    c2 = pltpu.async_copy(em_h.at[midx], mrows, sem)
    c3 = pltpu.async_copy(bu16_h.at[uidx16], buv, sem)
    c4 = pltpu.async_copy(bm16_h.at[midx16], bmv, sem)
    c1.wait()
    c2.wait()
    c3.wait()
    c4.wait()
    pltpu.sync_copy(urows, out_u.at[pl.ds(base, BPW)])
    pltpu.sync_copy(mrows, out_m.at[pl.ds(base, BPW)])
    pltpu.sync_copy(buv, out_bu.at[pl.ds(base, BPW)])
    pltpu.sync_copy(bmv, out_bm.at[pl.ds(base, BPW)])


@functools.cache
def _sc_gather():
    return pl.kernel(
        _sc_gather_body,
        out_type=[
            jax.ShapeDtypeStruct((B, U_DIM), jnp.float32),
            jax.ShapeDtypeStruct((B, M_DIM), jnp.float32),
            jax.ShapeDtypeStruct((B, 16), jnp.float32),
            jax.ShapeDtypeStruct((B, 16), jnp.float32),
        ],
        mesh=plsc.VectorSubcoreMesh(core_axis_name="c", subcore_axis_name="s",
                                    num_cores=_NC, num_subcores=_NS),
        compiler_params=pltpu.CompilerParams(use_tc_tiling_on_sc=False),
        scratch_types=[
            pltpu.VMEM((BPW,), jnp.int32),
            pltpu.VMEM((BPW,), jnp.int32),
            pltpu.VMEM((BPW,), jnp.int32),
            pltpu.VMEM((BPW,), jnp.int32),
            pltpu.VMEM((BPW, U_DIM), jnp.float32),
            pltpu.VMEM((BPW, M_DIM), jnp.float32),
            pltpu.VMEM((BPW, 16), jnp.float32),
            pltpu.VMEM((BPW, 16), jnp.float32),
            pltpu.SemaphoreType.DMA,
        ],
    )


BB = 2048


def _tc_dense_body(uid_ref, mid_ref, age_ref, gen_ref, catsT_ref,
                   ru_ref, rm_ref, bu16_ref, bm16_ref,
                   ea_ref, eg_ref, ec_ref, uw_ref, ub_ref, mw_ref, mb_ref,
                   out_ref):
    f32 = jnp.float32
    uw = uw_ref[...]
    mw = mw_ref[...]
    a_age = jnp.dot(ea_ref[...], uw[64:80, :], preferred_element_type=f32)
    a_gen = jnp.dot(eg_ref[...], uw[80:96, :], preferred_element_type=f32)
    a_cat = jnp.dot(ec_ref[...], mw[64:96, :], preferred_element_type=f32)

    age = age_ref[...]
    gen = gen_ref[...]
    aoh = (age[:, None] == lax.broadcasted_iota(jnp.int32, (BB, 8), 1)).astype(f32)
    goh = (gen[:, None] == lax.broadcasted_iota(jnp.int32, (BB, 4), 1)).astype(f32)

    uv = (jnp.dot(ru_ref[...], uw[:64, :], preferred_element_type=f32)
          + jnp.dot(aoh, a_age, preferred_element_type=f32)
          + jnp.dot(goh, a_gen, preferred_element_type=f32)
          + ub_ref[...][None, :])

    iota32 = lax.broadcasted_iota(jnp.int32, (BB, 32), 1)
    coh = jnp.zeros((BB, 32), dtype=f32)
    for j in range(8):
        coh = coh + (catsT_ref[j, :][:, None] == iota32).astype(f32)
    coh = coh * (iota32 != 0).astype(f32)
    cnt = jnp.maximum(jnp.sum(coh, axis=1, keepdims=True), 1.0)
    coh = coh / cnt

    mv = (jnp.dot(rm_ref[...], mw[:64, :], preferred_element_type=f32)
          + jnp.dot(coh, a_cat, preferred_element_type=f32)
          + mb_ref[...][None, :])

    iota16 = lax.broadcasted_iota(jnp.int32, (BB, 16), 1)
    uoh = ((uid_ref[...] & 15)[:, None] == iota16).astype(f32)
    moh = ((mid_ref[...] & 15)[:, None] == iota16).astype(f32)
    bu = jnp.sum(bu16_ref[...] * uoh, axis=1)
    bm = jnp.sum(bm16_ref[...] * moh, axis=1)

    s = jnp.sum(uv * mv, axis=1) + bu + bm
    p = 1.0 / (1.0 + jnp.exp(-s))
    out_ref[...] = p * (1.0 + 2 * 0.1) - 0.1


def _tc_dense(uid, mid, user_age, user_gender, catsT, raw_user, raw_movie,
              bu16, bm16, emb_age, emb_gender, emb_movie_cats,
              user_fc_w, user_fc_b, movie_fc_w, movie_fc_b):
    grid = (B // BB,)
    full = lambda i: (0, 0)
    return pl.pallas_call(
        _tc_dense_body,
        grid=grid,
        in_specs=[
            pl.BlockSpec((BB,), lambda i: (i,)),
            pl.BlockSpec((BB,), lambda i: (i,)),
            pl.BlockSpec((BB,), lambda i: (i,)),
            pl.BlockSpec((BB,), lambda i: (i,)),
            pl.BlockSpec((8, BB), lambda i: (0, i)),
            pl.BlockSpec((BB, U_DIM), lambda i: (i, 0)),
            pl.BlockSpec((BB, M_DIM), lambda i: (i, 0)),
            pl.BlockSpec((BB, 16), lambda i: (i, 0)),
            pl.BlockSpec((BB, 16), lambda i: (i, 0)),
            pl.BlockSpec((8, 16), full),
            pl.BlockSpec((4, 16), full),
            pl.BlockSpec((32, 32), full),
            pl.BlockSpec((96, 20), full),
            pl.BlockSpec((20,), lambda i: (0,)),
            pl.BlockSpec((96, 20), full),
            pl.BlockSpec((20,), lambda i: (0,)),
        ],
        out_specs=pl.BlockSpec((BB,), lambda i: (i,)),
        out_shape=jax.ShapeDtypeStruct((B,), jnp.float32),
    )(uid, mid, user_age, user_gender, catsT, raw_user, raw_movie,
      bu16, bm16, emb_age, emb_gender, emb_movie_cats,
      user_fc_w, user_fc_b, movie_fc_w, movie_fc_b)


def kernel(user_id, user_age, user_gender, movie_id, movie_categories,
           emb_users, emb_movies, emb_movie_cats, emb_age, emb_gender,
           bias_user, bias_movie, user_fc_w, user_fc_b, movie_fc_w, movie_fc_b):
    uid = user_id.astype(jnp.int32)
    mid = movie_id.astype(jnp.int32)
    bu_view = bias_user.reshape(-1).reshape(-1, 16)
    bm_view = bias_movie.reshape(-1).reshape(-1, 16)
    raw_user, raw_movie, bu16, bm16 = _sc_gather()(
        uid, mid, uid >> 4, mid >> 4, emb_users, emb_movies, bu_view, bm_view)
    catsT = movie_categories.astype(jnp.int32).T
    return _tc_dense(uid, mid,
                     user_age.astype(jnp.int32), user_gender.astype(jnp.int32),
                     catsT, raw_user, raw_movie, bu16, bm16,
                     emb_age, emb_gender, emb_movie_cats,
                     user_fc_w, user_fc_b, movie_fc_w, movie_fc_b)
